# Initial kernel scaffold; baseline (speedup 1.0000x reference)
import jax
import jax.numpy as jnp
from jax.experimental import pallas as pl


def kernel(x, table):
    raise NotImplementedError("write your pallas kernel here")


# speedup vs baseline: 1.2842x; 1.2842x over previous
import functools

import numpy as np
import jax
import jax.numpy as jnp
from jax import lax
from jax.experimental import pallas as pl
from jax.experimental.pallas import tpu as pltpu
from jax.experimental.pallas import tpu_sc as plsc

_VOCAB = 1000000
_EMBED = 64
_SEQ = 50
_BATCH = 16384

_NC = 2
_NS = 16
_NW = _NC * _NS

_CHUNK_B = 16
_CHUNK_R = _CHUNK_B * _SEQ
_BATCH_PER_W = _BATCH // _NW
_NCHUNKS = _BATCH_PER_W // _CHUNK_B


def _positional_encoding(seq_len, d_model):
    pos = np.arange(seq_len)[:, np.newaxis]
    i = np.arange(d_model)[np.newaxis, :]
    angle_rates = 1.0 / np.power(10000, 2 * (i // 2) / np.float32(d_model))
    angle_rads = pos * angle_rates
    angle_rads[:, 0::2] = np.sin(angle_rads[:, 0::2])
    angle_rads[:, 1::2] = np.cos(angle_rads[:, 1::2])
    return angle_rads.astype(np.float32)


def _body(x_hbm, pos_hbm, table_hbm, out_hbm, idx_v, pos_v, rows_v, sem):
    c = lax.axis_index("c")
    s = lax.axis_index("s")
    wid = s * _NC + c

    pltpu.sync_copy(pos_hbm, pos_v)

    def chunk(ci, carry):
        bbase = wid * _BATCH_PER_W + ci * _CHUNK_B
        pltpu.sync_copy(x_hbm.at[pl.ds(bbase, _CHUNK_B)], idx_v)
        copies = []
        for b in range(_CHUNK_B):
            copies.append(pltpu.async_copy(
                table_hbm.at[idx_v.at[b]],
                rows_v.at[pl.ds(b * _SEQ, _SEQ)],
                sem))
        for cp in copies:
            cp.wait()

        def add_l(l, _):
            p0 = pos_v[l, pl.ds(0, 16)]
            p1 = pos_v[l, pl.ds(16, 16)]
            p2 = pos_v[l, pl.ds(32, 16)]
            p3 = pos_v[l, pl.ds(48, 16)]

            def add_b(b, _):
                r = b * _SEQ + l
                rows_v[r, pl.ds(0, 16)] = rows_v[r, pl.ds(0, 16)] + p0
                rows_v[r, pl.ds(16, 16)] = rows_v[r, pl.ds(16, 16)] + p1
                rows_v[r, pl.ds(32, 16)] = rows_v[r, pl.ds(32, 16)] + p2
                rows_v[r, pl.ds(48, 16)] = rows_v[r, pl.ds(48, 16)] + p3
                return 0

            return lax.fori_loop(0, _CHUNK_B, add_b, 0)

        lax.fori_loop(0, _SEQ, add_l, 0)

        pltpu.sync_copy(rows_v, out_hbm.at[pl.ds(bbase * _SEQ, _CHUNK_R)])
        return carry

    lax.fori_loop(0, _NCHUNKS, chunk, 0)


@functools.partial(jax.jit, static_argnames=())
def kernel(x, table):
    pos = jnp.asarray(_positional_encoding(_SEQ, _EMBED))
    mesh = plsc.VectorSubcoreMesh(core_axis_name="c", subcore_axis_name="s")
    run = pl.kernel(
        _body,
        out_type=jax.ShapeDtypeStruct((_BATCH * _SEQ, _EMBED), jnp.float32),
        mesh=mesh,
        scratch_types=[
            pltpu.VMEM((_CHUNK_B, _SEQ), jnp.int32),
            pltpu.VMEM((_SEQ, _EMBED), jnp.float32),
            pltpu.VMEM((_CHUNK_R, _EMBED), jnp.float32),
            pltpu.SemaphoreType.DMA,
        ],
        compiler_params=pltpu.CompilerParams(use_tc_tiling_on_sc=False),
    )
    out_flat = run(x, pos, table)
    return out_flat.reshape(_BATCH, _SEQ, _EMBED)


# speedup vs baseline: 1.8595x; 1.4480x over previous
import functools

import numpy as np
import jax
import jax.numpy as jnp
from jax import lax
from jax.experimental import pallas as pl
from jax.experimental.pallas import tpu as pltpu
from jax.experimental.pallas import tpu_sc as plsc

_VOCAB = 1000000
_EMBED = 64
_SEQ = 50
_BATCH = 16384

_NC = 2
_NS = 16
_NW = _NC * _NS

_IDXW = 100
_CHUNK_G = 4
_CHUNK_R = _CHUNK_G * _IDXW
_ROWS_PER_W = _BATCH * _SEQ // _NW
_NCHUNKS = _ROWS_PER_W // _CHUNK_R
_IDXROWS_PER_W = _ROWS_PER_W // _IDXW


def _positional_encoding(seq_len, d_model):
    pos = np.arange(seq_len)[:, np.newaxis]
    i = np.arange(d_model)[np.newaxis, :]
    angle_rates = 1.0 / np.power(10000, 2 * (i // 2) / np.float32(d_model))
    angle_rads = pos * angle_rates
    angle_rads[:, 0::2] = np.sin(angle_rads[:, 0::2])
    angle_rads[:, 1::2] = np.cos(angle_rads[:, 1::2])
    return angle_rads.astype(np.float32)


def _body(x_hbm, pos_hbm, table_hbm, out_hbm,
          idx0, idx1, pos_v, rows0, rows1, outb0, outb1,
          gsem0, gsem1, osem0, osem1, isem0, isem1):
    c = lax.axis_index("c")
    s = lax.axis_index("s")
    wid = s * _NC + c
    idx_base = wid * _IDXROWS_PER_W
    row_base = wid * _ROWS_PER_W

    pltpu.sync_copy(pos_hbm, pos_v)

    def fire_gathers(chunk, idx_v, rows_v, sem):
        for g in range(_CHUNK_G):
            pltpu.async_copy(
                table_hbm.at[idx_v.at[g]],
                rows_v.at[pl.ds(g * _IDXW, _IDXW)],
                sem)

    def add_pos(rows_v, outb_v):
        def add_l(l, _):
            p0 = pos_v[l, pl.ds(0, 16)]
            p1 = pos_v[l, pl.ds(16, 16)]
            p2 = pos_v[l, pl.ds(32, 16)]
            p3 = pos_v[l, pl.ds(48, 16)]

            @plsc.parallel_loop(0, _CHUNK_R // _SEQ, unroll=4)
            def add_b(b):
                r = b * _SEQ + l
                outb_v[r, pl.ds(0, 16)] = rows_v[r, pl.ds(0, 16)] + p0
                outb_v[r, pl.ds(16, 16)] = rows_v[r, pl.ds(16, 16)] + p1
                outb_v[r, pl.ds(32, 16)] = rows_v[r, pl.ds(32, 16)] + p2
                outb_v[r, pl.ds(48, 16)] = rows_v[r, pl.ds(48, 16)] + p3

            return 0

        lax.fori_loop(0, _SEQ, add_l, 0)

    pltpu.sync_copy(x_hbm.at[pl.ds(idx_base, _CHUNK_G)], idx0)
    pltpu.sync_copy(x_hbm.at[pl.ds(idx_base + _CHUNK_G, _CHUNK_G)], idx1)
    fire_gathers(0, idx0, rows0, gsem0)
    fire_gathers(1, idx1, rows1, gsem1)

    def loop_body(j, carry):
        @pl.when(j >= 1)
        def _():
            pltpu.make_async_copy(outb0, out_hbm.at[pl.ds(0, _CHUNK_R)], osem0).wait()
            pltpu.make_async_copy(outb1, out_hbm.at[pl.ds(0, _CHUNK_R)], osem1).wait()

        for par, (idx_v, rows_v, outb_v, gsem, osem, isem) in enumerate((
                (idx0, rows0, outb0, gsem0, osem0, isem0),
                (idx1, rows1, outb1, gsem1, osem1, isem1))):
            ch = 2 * j + par
            pltpu.make_async_copy(
                out_hbm.at[pl.ds(0, _CHUNK_R)], rows_v, gsem).wait()
            nxt = lax.rem(ch + 2, _NCHUNKS)
            ih = pltpu.async_copy(
                x_hbm.at[pl.ds(idx_base + nxt * _CHUNK_G, _CHUNK_G)],
                idx_v, isem)
            add_pos(rows_v, outb_v)
            pltpu.async_copy(
                outb_v, out_hbm.at[pl.ds(row_base + ch * _CHUNK_R, _CHUNK_R)],
                osem)
            ih.wait()
            fire_gathers(nxt, idx_v, rows_v, gsem)
        return carry

    lax.fori_loop(0, _NCHUNKS // 2, loop_body, 0)

    pltpu.make_async_copy(outb0, out_hbm.at[pl.ds(0, _CHUNK_R)], osem0).wait()
    pltpu.make_async_copy(outb1, out_hbm.at[pl.ds(0, _CHUNK_R)], osem1).wait()
    pltpu.make_async_copy(out_hbm.at[pl.ds(0, _CHUNK_R)], rows0, gsem0).wait()
    pltpu.make_async_copy(out_hbm.at[pl.ds(0, _CHUNK_R)], rows1, gsem1).wait()


@functools.partial(jax.jit, static_argnames=())
def kernel(x, table):
    pos = jnp.asarray(_positional_encoding(_SEQ, _EMBED))
    x2 = x.reshape(_BATCH * _SEQ // _IDXW, _IDXW)
    mesh = plsc.VectorSubcoreMesh(core_axis_name="c", subcore_axis_name="s")
    run = pl.kernel(
        _body,
        out_type=jax.ShapeDtypeStruct((_BATCH * _SEQ, _EMBED), jnp.float32),
        mesh=mesh,
        scratch_types=[
            pltpu.VMEM((_CHUNK_G, _IDXW), jnp.int32),
            pltpu.VMEM((_CHUNK_G, _IDXW), jnp.int32),
            pltpu.VMEM((_SEQ, _EMBED), jnp.float32),
            pltpu.VMEM((_CHUNK_R, _EMBED), jnp.float32),
            pltpu.VMEM((_CHUNK_R, _EMBED), jnp.float32),
            pltpu.VMEM((_CHUNK_R, _EMBED), jnp.float32),
            pltpu.VMEM((_CHUNK_R, _EMBED), jnp.float32),
            pltpu.SemaphoreType.DMA,
            pltpu.SemaphoreType.DMA,
            pltpu.SemaphoreType.DMA,
            pltpu.SemaphoreType.DMA,
            pltpu.SemaphoreType.DMA,
            pltpu.SemaphoreType.DMA,
        ],
        compiler_params=pltpu.CompilerParams(use_tc_tiling_on_sc=False),
    )
    out_flat = run(x2, pos, table)
    return out_flat.reshape(_BATCH, _SEQ, _EMBED)
